# Initial kernel scaffold; baseline (speedup 1.0000x reference)
import jax
import jax.numpy as jnp
from jax.experimental import pallas as pl


def kernel(xcorr):
    raise NotImplementedError("write your pallas kernel here")


# speedup vs baseline: 33.3587x; 33.3587x over previous
import functools

import jax
import jax.numpy as jnp
from jax.experimental import pallas as pl


def _body(W, nlag, x_ref, nb_ref, val_ref, idx_ref):
    x = x_ref[...]
    R = x.shape[0]
    xl = jnp.concatenate([x[:, :1], x[:, :-1]], axis=1)
    xr = jnp.concatenate([x[:, 1:], x[:, -1:]], axis=1)
    ispeak = (x >= xl) & (x >= xr)
    scores = jnp.where(ispeak, x, 0.0)

    iota = jax.lax.broadcasted_iota(jnp.int32, (R, W), 1)
    neg_inf = jnp.float32(-jnp.inf)

    def top1(s):
        m = jnp.max(s, axis=1, keepdims=True)
        i = jnp.min(jnp.where(s == m, iota, W), axis=1, keepdims=True)
        return m, i

    m1, i1 = top1(scores)
    s2 = jnp.where(iota == i1, neg_inf, scores)
    m2, i2 = top1(s2)
    s3 = jnp.where(iota == i2, neg_inf, s2)
    m3, i3 = top1(s3)

    sel1 = iota == i1
    zero = jnp.float32(0.0)
    n0 = jnp.sum(jnp.where(sel1, xl, zero), axis=1, keepdims=True)
    n1 = jnp.sum(jnp.where(sel1, x, zero), axis=1, keepdims=True)
    n2 = jnp.sum(jnp.where(sel1, xr, zero), axis=1, keepdims=True)

    nb_ref[...] = jnp.concatenate([n0, n1, n2], axis=1)
    val_ref[...] = jnp.concatenate([m1, m2, m3], axis=1)
    idx_ref[...] = jnp.concatenate([i1, i2, i3], axis=1) - nlag


@jax.jit
def kernel(xcorr):
    B, C, H, W = xcorr.shape
    N = B * C * H
    nlag = W // 2
    R = 256
    x2 = xcorr.reshape(N, W)
    grid = (N // R,)
    out_shape = [
        jax.ShapeDtypeStruct((N, 3), jnp.float32),
        jax.ShapeDtypeStruct((N, 3), jnp.float32),
        jax.ShapeDtypeStruct((N, 3), jnp.int32),
    ]
    out_spec = pl.BlockSpec((R, 3), lambda i: (i, 0))
    nb, vals, idxs = pl.pallas_call(
        functools.partial(_body, W, nlag),
        grid=grid,
        in_specs=[pl.BlockSpec((R, W), lambda i: (i, 0))],
        out_specs=[out_spec, out_spec, out_spec],
        out_shape=out_shape,
    )(x2)
    shp = (B, C, H, 3)
    return nb.reshape(shp), vals.reshape(shp), idxs.reshape(shp)


# speedup vs baseline: 60.2229x; 1.8053x over previous
import functools

import jax
import jax.numpy as jnp
from jax.experimental import pallas as pl
from jax.experimental.pallas import tpu as pltpu

_LANES = 128
_ROWS = 256
_SUB = 8


def _body(W, nlag, x_ref, nb_ref, val_ref, idx_ref,
          m1_s, m2_s, m3_s, i1_s, i2_s, i3_s, nl_s, nr_s):
    R = nb_ref.shape[0]
    nchunk = W // _LANES
    neg_inf = jnp.float32(-jnp.inf)
    zero = jnp.float32(0.0)
    big = jnp.int32(1 << 30)

    lane8 = jax.lax.broadcasted_iota(jnp.int32, (_SUB, _LANES), 1)
    minf = jnp.full((_SUB, _LANES), neg_inf)
    zi = jnp.zeros((_SUB, _LANES), jnp.int32)
    zf = jnp.zeros((_SUB, _LANES), jnp.float32)

    for g in range(R // _SUB):
        r0 = g * _SUB
        rows = pl.ds(r0, _SUB)
        m1, m2, m3 = minf, minf, minf
        i1, i2, i3 = zi, zi, zi
        nl, nr = zf, zf
        xc = x_ref[rows, 0:_LANES]
        last = xc[:, :1]
        for c in range(nchunk):
            if c + 1 < nchunk:
                xn = x_ref[rows, (c + 1) * _LANES:(c + 2) * _LANES]
                first_next = xn[:, :1]
            else:
                xn = None
                first_next = xc[:, -1:]
            xl = jnp.concatenate([last, xc[:, :-1]], axis=1)
            xr = jnp.concatenate([xc[:, 1:], first_next], axis=1)
            sc = jnp.where(xc >= jnp.maximum(xl, xr), xc, zero)
            ivec = lane8 + (c * _LANES)
            b1 = sc > m1
            b2 = sc > m2
            b3 = sc > m3
            m3 = jnp.where(b3, jnp.where(b2, m2, sc), m3)
            i3 = jnp.where(b3, jnp.where(b2, i2, ivec), i3)
            m2 = jnp.where(b2, jnp.where(b1, m1, sc), m2)
            i2 = jnp.where(b2, jnp.where(b1, i1, ivec), i2)
            m1 = jnp.where(b1, sc, m1)
            i1 = jnp.where(b1, ivec, i1)
            nl = jnp.where(b1, xl, nl)
            nr = jnp.where(b1, xr, nr)
            last = xc[:, -1:]
            xc = xn
        m1_s[rows, :] = m1
        m2_s[rows, :] = m2
        m3_s[rows, :] = m3
        i1_s[rows, :] = i1
        i2_s[rows, :] = i2
        i3_s[rows, :] = i3
        nl_s[rows, :] = nl
        nr_s[rows, :] = nr

    m1 = m1_s[...]
    m2 = m2_s[...]
    m3 = m3_s[...]
    i1 = i1_s[...]
    i2 = i2_s[...]
    i3 = i3_s[...]
    v1 = jnp.max(m1, axis=1, keepdims=True)
    idx1 = jnp.min(jnp.where(m1 == v1, i1, big), axis=1, keepdims=True)
    cond1 = i1 == idx1
    n0 = jnp.sum(jnp.where(cond1, nl_s[...], zero), axis=1, keepdims=True)
    n2 = jnp.sum(jnp.where(cond1, nr_s[...], zero), axis=1, keepdims=True)
    m1 = jnp.where(cond1, neg_inf, m1)

    def next_best(m1, m2, m3):
        v = jnp.max(jnp.maximum(m1, jnp.maximum(m2, m3)), axis=1,
                    keepdims=True)
        cand = jnp.minimum(
            jnp.where(m1 == v, i1, big),
            jnp.minimum(jnp.where(m2 == v, i2, big),
                        jnp.where(m3 == v, i3, big)))
        idx = jnp.min(cand, axis=1, keepdims=True)
        return v, idx, (jnp.where(i1 == idx, neg_inf, m1),
                        jnp.where(i2 == idx, neg_inf, m2),
                        jnp.where(i3 == idx, neg_inf, m3))

    v2, idx2, (m1, m2, m3) = next_best(m1, m2, m3)
    v3, idx3, _ = next_best(m1, m2, m3)

    nb_ref[...] = jnp.concatenate([n0, v1, n2], axis=1)
    val_ref[...] = jnp.concatenate([v1, v2, v3], axis=1)
    idx_ref[...] = jnp.concatenate([idx1, idx2, idx3], axis=1) - nlag


@jax.jit
def kernel(xcorr):
    B, C, H, W = xcorr.shape
    N = B * C * H
    nlag = W // 2
    R = _ROWS
    x2 = xcorr.reshape(N, W)
    grid = (N // R,)
    out_shape = [
        jax.ShapeDtypeStruct((N, 3), jnp.float32),
        jax.ShapeDtypeStruct((N, 3), jnp.float32),
        jax.ShapeDtypeStruct((N, 3), jnp.int32),
    ]
    out_spec = pl.BlockSpec((R, 3), lambda i: (i, 0))
    f32s = functools.partial(pltpu.VMEM, (R, _LANES), jnp.float32)
    i32s = functools.partial(pltpu.VMEM, (R, _LANES), jnp.int32)
    nb, vals, idxs = pl.pallas_call(
        functools.partial(_body, W, nlag),
        grid=grid,
        in_specs=[pl.BlockSpec((R, W), lambda i: (i, 0))],
        out_specs=[out_spec, out_spec, out_spec],
        out_shape=out_shape,
        scratch_shapes=[f32s(), f32s(), f32s(), i32s(), i32s(), i32s(),
                        f32s(), f32s()],
    )(x2)
    shp = (B, C, H, 3)
    return nb.reshape(shp), vals.reshape(shp), idxs.reshape(shp)
